# baseline (speedup 1.0000x reference)
import functools
from collections.abc import Sequence

import jax
import jax.numpy as jnp
from jax.experimental import pallas as pl
import jax.experimental.pallas.tpu as pltpu
from jax.sharding import Mesh, PartitionSpec as P

N, F_IN, H, C = 10000, 128, 128, 64
NDEV = 2
NL = N // NDEV
BS = 200


def _stage1_kernel(adj_ref, x_ref, w1_ref, w2_ref, b1_ref, t_ref, s1_ref):
    @pl.when(pl.program_id(0) == 0)
    def _():
        s1_ref[...] = jnp.dot(x_ref[...], w1_ref[...],
                              preferred_element_type=jnp.float32)

    acc = jnp.dot(adj_ref[...], s1_ref[...],
                  preferred_element_type=jnp.float32)
    h1 = jnp.maximum(acc + b1_ref[...], 0.0)
    t_ref[...] = jnp.dot(h1, w2_ref[...], preferred_element_type=jnp.float32)


def _stage2_kernel(adj_ref, t_ref, b2_ref, out_ref):
    z = jnp.dot(adj_ref[...], t_ref[...],
                preferred_element_type=jnp.float32) + b2_ref[...]
    zs = z - jnp.max(z, axis=1, keepdims=True)
    out_ref[...] = zs - jnp.log(jnp.sum(jnp.exp(zs), axis=1, keepdims=True))


def _per_core(adj_local, x, W1, b1, W2, b2):
    grid = (NL // BS,)
    strip = pl.BlockSpec((BS, N), lambda i: (i, 0))
    full = lambda shape: pl.BlockSpec(shape, lambda i: (0, 0))

    t_local = pl.pallas_call(
        _stage1_kernel,
        grid=grid,
        in_specs=[strip, full((N, F_IN)), full((F_IN, H)), full((H, C)),
                  full((1, H))],
        out_specs=pl.BlockSpec((BS, C), lambda i: (i, 0)),
        out_shape=jax.ShapeDtypeStruct((NL, C), jnp.float32),
        scratch_shapes=[pltpu.VMEM((N, H), jnp.float32)],
    )(adj_local, x, W1, W2, b1.reshape(1, H))

    t_full = jax.lax.all_gather(t_local, 'd', axis=0, tiled=True)

    out_local = pl.pallas_call(
        _stage2_kernel,
        grid=grid,
        in_specs=[strip, full((N, C)), full((1, C))],
        out_specs=pl.BlockSpec((BS, C), lambda i: (i, 0)),
        out_shape=jax.ShapeDtypeStruct((NL, C), jnp.float32),
    )(adj_local, t_full, b2.reshape(1, C))
    return out_local


@jax.jit
def kernel(x, adj, W1, b1, W2, b2):
    mesh = Mesh(jax.devices()[:NDEV], ('d',))
    f = jax.shard_map(
        _per_core, mesh=mesh,
        in_specs=(P('d', None), P(None, None), P(None, None), P(None),
                  P(None, None), P(None)),
        out_specs=P('d', None), check_vma=False)
    return f(adj, x, W1, b1, W2, b2)


# speedup vs baseline: 4.1042x; 4.1042x over previous
import functools

import numpy as np
import jax
import jax.numpy as jnp
from jax.experimental import pallas as pl
import jax.experimental.pallas.tpu as pltpu

N, F_IN, H, C = 10000, 128, 128, 64
KR, BMR = 5, 2048
KC, BN = 5, 2048
G = BN // BMR
NP = KR * BMR
EDGE_R = N - (KR - 1) * BMR
EDGE_C = N - (KC - 1) * BN
HB = BMR // 2
R = H + C
OA = R

_COL_AR, _COL_AC, _COL_IB, _COL_L1, _COL_L1I, _COL_FIN, _COL_L2R, _COL_FL2, \
    _COL_L2I, _COL_EMIT, _COL_OUTB = range(11)


def _build_schedule() -> np.ndarray:
    rows = []
    seen_l2 = set()

    def add(ar, ac, ib, l1, l1init, fin, l2res, finl2, emit, outb):
        l2init = 1 if ((l2res or finl2) and ib not in seen_l2) else 0
        if l2res or finl2:
            seen_l2.add(ib)
        rows.append(
            [ar, ac, ib, l1, l1init, fin, l2res, finl2, l2init, emit, 0])

    def ready(c, i):
        return G * c + G - 1 < i

    def diag_ok(c, i):
        return c == i // G and i % G == G - 1

    for i in range(KR):
        cd = i // G
        for p in range(KC):
            c = (cd + 1 + p) % KC
            fin = 1 if p == KC - 1 else 0
            add(i, c, i, 1, 1 if p == 0 else 0, fin,
                1 if ready(c, i) else 0,
                1 if (fin and diag_ok(c, i)) else 0, 0, 0)
    for i in range(KR):
        cset = [c for c in range(KC) if not (ready(c, i) or diag_ok(c, i))]
        for idx, c in enumerate(cset):
            add(i, c, i, 0, 0, 0, 1, 0, 1 if idx == len(cset) - 1 else 0, i)
        if not cset:
            add(KR - 2, KC - 1, i, 0, 0, 0, 0, 0, 1, i)

    arr = np.asarray(rows, dtype=np.int32)
    outb = KR - 1
    for s in range(arr.shape[0] - 1, -1, -1):
        if arr[s, _COL_EMIT]:
            outb = arr[s, _COL_IB]
        arr[s, _COL_OUTB] = outb
    return arr


_SCHEDULE = _build_schedule()
_STEPS = _SCHEDULE.shape[0]


def _gcn_kernel(sref, adjt_ref, adjb_ref, x_ref, w1_ref, w2_ref, b1_ref,
                b2_ref, out_ref, rhs_ref, acc_ref):
    step = pl.program_id(0)
    ib = sref[step, _COL_IB]
    jb = sref[step, _COL_AC]

    @pl.when(step == 0)
    def _():
        rhs_ref[pl.ds(0, N), pl.ds(0, H)] = jnp.dot(
            x_ref[...], w1_ref[...], preferred_element_type=jnp.float32)
        rhs_ref[pl.ds(N, NP - N), pl.ds(0, H)] = jnp.zeros(
            (NP - N, H), jnp.float32)

    def compute():
        rhs_slice = rhs_ref[pl.ds(jb * BN, BN), pl.ds(0, R)]
        res_t = jnp.dot(adjt_ref[...], rhs_slice,
                        preferred_element_type=jnp.float32)
        res_b = jnp.dot(adjb_ref[...], rhs_slice,
                        preferred_element_type=jnp.float32)

        @pl.when(sref[step, _COL_L1] == 1)
        def _():
            @pl.when(sref[step, _COL_L1I] == 1)
            def _():
                acc_ref[pl.ds(0, HB), :] = res_t[:, 0:H]
                acc_ref[pl.ds(HB, HB), :] = res_b[:, 0:H]

            @pl.when(sref[step, _COL_L1I] == 0)
            def _():
                acc_ref[pl.ds(0, HB), :] += res_t[:, 0:H]
                acc_ref[pl.ds(HB, HB), :] += res_b[:, 0:H]

        def oacc_update(c2_t, c2_b):
            @pl.when(sref[step, _COL_L2I] == 1)
            def _():
                rhs_ref[pl.ds(ib * BMR, HB), pl.ds(OA, C)] = c2_t
                rhs_ref[pl.ds(ib * BMR + HB, HB), pl.ds(OA, C)] = c2_b

            @pl.when(sref[step, _COL_L2I] == 0)
            def _():
                rhs_ref[pl.ds(ib * BMR, HB), pl.ds(OA, C)] += c2_t
                rhs_ref[pl.ds(ib * BMR + HB, HB), pl.ds(OA, C)] += c2_b

        @pl.when(sref[step, _COL_FIN] == 1)
        def _():
            h1 = jnp.maximum(acc_ref[...] + b1_ref[...], 0.0)
            tv = jnp.dot(h1, w2_ref[...], preferred_element_type=jnp.float32)
            row_limit = jnp.where(ib == KR - 1, EDGE_R, BMR)
            row_ok = jax.lax.broadcasted_iota(
                jnp.int32, (BMR, C), 0) < row_limit
            rhs_ref[pl.ds(ib * BMR, BMR), pl.ds(H, C)] = jnp.where(
                row_ok, tv, 0.0)

            @pl.when(sref[step, _COL_FL2] == 1)
            def _():
                tslice = rhs_ref[pl.ds(jb * BN, BN), pl.ds(H, C)]
                oacc_update(
                    jnp.dot(adjt_ref[...], tslice,
                            preferred_element_type=jnp.float32),
                    jnp.dot(adjb_ref[...], tslice,
                            preferred_element_type=jnp.float32))

        @pl.when(sref[step, _COL_L2R] == 1)
        def _():
            oacc_update(res_t[:, H:R], res_b[:, H:R])

    compute()

    @pl.when(sref[step, _COL_EMIT] == 1)
    def _():
        z = rhs_ref[pl.ds(ib * BMR, BMR), pl.ds(OA, C)] + b2_ref[...]
        zs = z - jnp.max(z, axis=1, keepdims=True)
        out_ref[...] = zs - jnp.log(
            jnp.sum(jnp.exp(zs), axis=1, keepdims=True))


@jax.jit
def kernel(x, adj, W1, b1, W2, b2):
    grid_spec = pltpu.PrefetchScalarGridSpec(
        num_scalar_prefetch=1,
        grid=(_STEPS,),
        in_specs=[
            pl.BlockSpec((HB, BN),
                         lambda s, sref: (2 * sref[s, _COL_AR],
                                          sref[s, _COL_AC]),
                         pipeline_mode=pl.Buffered(buffer_count=2)),
            pl.BlockSpec((HB, BN),
                         lambda s, sref: (2 * sref[s, _COL_AR] + 1,
                                          sref[s, _COL_AC]),
                         pipeline_mode=pl.Buffered(buffer_count=2)),
            pl.BlockSpec((N, F_IN), lambda s, sref: (0, 0)),
            pl.BlockSpec((F_IN, H), lambda s, sref: (0, 0)),
            pl.BlockSpec((H, C), lambda s, sref: (0, 0)),
            pl.BlockSpec((1, H), lambda s, sref: (0, 0)),
            pl.BlockSpec((1, C), lambda s, sref: (0, 0)),
        ],
        out_specs=pl.BlockSpec((BMR, C),
                               lambda s, sref: (sref[s, _COL_OUTB], 0)),
        scratch_shapes=[
            pltpu.VMEM((NP, OA + C), jnp.float32),
            pltpu.VMEM((BMR, H), jnp.float32),
        ],
    )
    return pl.pallas_call(
        _gcn_kernel,
        grid_spec=grid_spec,
        out_shape=jax.ShapeDtypeStruct((N, C), jnp.float32),
    )(jnp.asarray(_SCHEDULE), adj, adj, x.astype(jnp.bfloat16),
      W1.astype(jnp.bfloat16), W2,
      b1.reshape(1, H), b2.reshape(1, C))
